# Initial kernel scaffold; baseline (speedup 1.0000x reference)
import jax
import jax.numpy as jnp
from jax.experimental import pallas as pl


def kernel(x, edge_index, batch, emb, W1, b1, W2, b2, linW, linb):
    raise NotImplementedError("write your pallas kernel here")


# speedup vs baseline: 16.2237x; 16.2237x over previous
import functools

import jax
import jax.numpy as jnp
from jax import lax
from jax.experimental import pallas as pl
from jax.experimental.pallas import tpu as pltpu
from jax.experimental.pallas import tpu_sc as plsc

N = 10000
E = 320000
H = 128
NUM_GRAPHS = 128
NUM_CLASSES = 10

N_PAD = 10240
E_CHUNK = 128
E_CHUNKS_PER_TILE = 80
E_PAD = 32 * E_CHUNKS_PER_TILE * E_CHUNK
ROWS_PER_TILE = N_PAD // 32
ROWS_PER_SUBCORE = N_PAD // 16

_MESH = plsc.VectorSubcoreMesh(core_axis_name="c", subcore_axis_name="s")


@functools.partial(
    pl.kernel,
    out_type=[
        jax.ShapeDtypeStruct((N_PAD, H), jnp.float32),
        jax.ShapeDtypeStruct((2, N_PAD, H), jnp.float32),
    ],
    mesh=_MESH,
    scratch_types=[
        pltpu.VMEM((80,), jnp.int32),
        pltpu.VMEM((80, H), jnp.float32),
        pltpu.VMEM((E_CHUNKS_PER_TILE, E_CHUNK), jnp.int32),
        pltpu.VMEM((E_CHUNK, H), jnp.float32),
        pltpu.VMEM_SHARED((N_PAD, H), jnp.float32),
        pltpu.SemaphoreType.DMA,
    ],
)
def _sc_embed_deg(emb_h, x_h, zeros_h, ones_h, dst_h, h0_h, degp_h,
                  idx_v, rows_v, didx_v, ones_v, acc_s, sem):
    c = lax.axis_index("c")
    s = lax.axis_index("s")
    wid = s * 2 + c

    pltpu.sync_copy(ones_h, ones_v)
    pltpu.sync_copy(zeros_h.at[pl.ds(s * ROWS_PER_SUBCORE, ROWS_PER_SUBCORE)],
                    acc_s.at[pl.ds(s * ROWS_PER_SUBCORE, ROWS_PER_SUBCORE)])
    plsc.subcore_barrier()

    def emb_chunk(t, _):
        base = wid * ROWS_PER_TILE + t * 80
        pltpu.sync_copy(x_h.at[pl.ds(base, 80)], idx_v)
        pltpu.async_copy(emb_h.at[idx_v], rows_v, sem).wait()
        pltpu.sync_copy(rows_v, h0_h.at[pl.ds(base, 80)])
        return 0
    lax.fori_loop(0, ROWS_PER_TILE // 80, emb_chunk, 0)

    pltpu.sync_copy(dst_h.at[wid], didx_v)

    def deg_chunk(j, _):
        pltpu.sync_copy(ones_v, acc_s.at[didx_v.at[j]], add=True)
        return 0
    lax.fori_loop(0, E_CHUNKS_PER_TILE, deg_chunk, 0)
    plsc.subcore_barrier()

    pltpu.sync_copy(
        acc_s.at[pl.ds(s * ROWS_PER_SUBCORE, ROWS_PER_SUBCORE)],
        degp_h.at[c, pl.ds(s * ROWS_PER_SUBCORE, ROWS_PER_SUBCORE)])


@functools.partial(
    pl.kernel,
    out_type=jax.ShapeDtypeStruct((2, N_PAD, H), jnp.float32),
    mesh=_MESH,
    scratch_types=[
        pltpu.VMEM((E_CHUNKS_PER_TILE, E_CHUNK), jnp.int32),
        pltpu.VMEM((E_CHUNKS_PER_TILE, E_CHUNK), jnp.int32),
        pltpu.VMEM((E_CHUNK, H), jnp.float32),
        pltpu.VMEM_SHARED((N_PAD, H), jnp.float32),
        pltpu.SemaphoreType.DMA,
    ],
)
def _sc_edge_scatter(g_h, zeros_h, src_h, dst_h, out_h,
                     sidx_v, didx_v, rows_v, acc_s, sem):
    c = lax.axis_index("c")
    s = lax.axis_index("s")
    wid = s * 2 + c

    pltpu.sync_copy(zeros_h.at[pl.ds(s * ROWS_PER_SUBCORE, ROWS_PER_SUBCORE)],
                    acc_s.at[pl.ds(s * ROWS_PER_SUBCORE, ROWS_PER_SUBCORE)])
    plsc.subcore_barrier()

    pltpu.sync_copy(src_h.at[wid], sidx_v)
    pltpu.sync_copy(dst_h.at[wid], didx_v)

    def chunk(j, _):
        pltpu.async_copy(g_h.at[sidx_v.at[j]], rows_v, sem).wait()
        pltpu.sync_copy(rows_v, acc_s.at[didx_v.at[j]], add=True)
        return 0
    lax.fori_loop(0, E_CHUNKS_PER_TILE, chunk, 0)
    plsc.subcore_barrier()

    pltpu.sync_copy(
        acc_s.at[pl.ds(s * ROWS_PER_SUBCORE, ROWS_PER_SUBCORE)],
        out_h.at[c, pl.ds(s * ROWS_PER_SUBCORE, ROWS_PER_SUBCORE)])


_BLK = 256
_GRID = N_PAD // _BLK


def _tc_layer1_body(h0_r, degp_r, w_r, g_r, dinv_r):
    deg = 1.0 + degp_r[0] + degp_r[1]
    dinv = lax.rsqrt(deg)
    g_r[...] = dinv * jnp.dot(h0_r[...], w_r[...],
                              preferred_element_type=jnp.float32)
    dinv_r[...] = dinv[:, 0:1]


def _tc_layer1(h0, degp, W1):
    return pl.pallas_call(
        _tc_layer1_body,
        grid=(_GRID,),
        in_specs=[
            pl.BlockSpec((_BLK, H), lambda i: (i, 0)),
            pl.BlockSpec((2, _BLK, H), lambda i: (0, i, 0)),
            pl.BlockSpec((H, H), lambda i: (0, 0)),
        ],
        out_specs=[
            pl.BlockSpec((_BLK, H), lambda i: (i, 0)),
            pl.BlockSpec((_BLK, 1), lambda i: (i, 0)),
        ],
        out_shape=[
            jax.ShapeDtypeStruct((N_PAD, H), jnp.float32),
            jax.ShapeDtypeStruct((N_PAD, 1), jnp.float32),
        ],
    )(h0, degp, W1)


def _tc_layer2_body(sp_r, g1_r, dinv_r, b_r, w_r, g2_r):
    h1 = jnp.maximum(
        dinv_r[...] * (sp_r[0] + sp_r[1] + g1_r[...]) + b_r[...], 0.0)
    g2_r[...] = dinv_r[...] * jnp.dot(h1, w_r[...],
                                      preferred_element_type=jnp.float32)


def _tc_layer2(Sp, g1, dinv, b1, W2):
    return pl.pallas_call(
        _tc_layer2_body,
        grid=(_GRID,),
        in_specs=[
            pl.BlockSpec((2, _BLK, H), lambda i: (0, i, 0)),
            pl.BlockSpec((_BLK, H), lambda i: (i, 0)),
            pl.BlockSpec((_BLK, 1), lambda i: (i, 0)),
            pl.BlockSpec((1, H), lambda i: (0, 0)),
            pl.BlockSpec((H, H), lambda i: (0, 0)),
        ],
        out_specs=pl.BlockSpec((_BLK, H), lambda i: (i, 0)),
        out_shape=jax.ShapeDtypeStruct((N_PAD, H), jnp.float32),
    )(Sp, g1, dinv, b1, W2)


def _tc_final_body(sp_r, g2_r, dinv_r, b_r, batch_r, w_r, bb_r, out_r,
                   pooled_s, cnt_s):
    i = pl.program_id(0)

    @pl.when(i == 0)
    def _():
        pooled_s[...] = jnp.zeros((NUM_GRAPHS, H), jnp.float32)
        cnt_s[...] = jnp.zeros((NUM_GRAPHS, H), jnp.float32)

    h2 = jnp.maximum(
        dinv_r[...] * (sp_r[0] + sp_r[1] + g2_r[...]) + b_r[...], 0.0)
    gids = lax.broadcasted_iota(jnp.int32, (_BLK, NUM_GRAPHS), 1)
    onehot = (batch_r[...] == gids).astype(jnp.float32)
    dn = (((0,), (0,)), ((), ()))
    pooled_s[...] += lax.dot_general(onehot, h2, dn,
                                     preferred_element_type=jnp.float32)
    cnt_s[...] += lax.dot_general(onehot, jnp.ones((_BLK, H), jnp.float32),
                                  dn, preferred_element_type=jnp.float32)

    @pl.when(i == _GRID - 1)
    def _():
        pooled = pooled_s[...] / jnp.maximum(cnt_s[...], 1.0)
        out_r[...] = jnp.dot(pooled, w_r[...],
                             preferred_element_type=jnp.float32) + bb_r[...]


def _tc_final(Sp, g2, dinv, b2, batch_p, linW_p, linb_p):
    return pl.pallas_call(
        _tc_final_body,
        grid=(_GRID,),
        in_specs=[
            pl.BlockSpec((2, _BLK, H), lambda i: (0, i, 0)),
            pl.BlockSpec((_BLK, H), lambda i: (i, 0)),
            pl.BlockSpec((_BLK, 1), lambda i: (i, 0)),
            pl.BlockSpec((1, H), lambda i: (0, 0)),
            pl.BlockSpec((_BLK, 1), lambda i: (i, 0)),
            pl.BlockSpec((H, H), lambda i: (0, 0)),
            pl.BlockSpec((1, H), lambda i: (0, 0)),
        ],
        out_specs=pl.BlockSpec((NUM_GRAPHS, H), lambda i: (0, 0)),
        out_shape=jax.ShapeDtypeStruct((NUM_GRAPHS, H), jnp.float32),
        scratch_shapes=[
            pltpu.VMEM((NUM_GRAPHS, H), jnp.float32),
            pltpu.VMEM((NUM_GRAPHS, H), jnp.float32),
        ],
    )(Sp, g2, dinv, b2, batch_p, linW_p, linb_p)


def kernel(x, edge_index, batch, emb, W1, b1, W2, b2, linW, linb):
    src = edge_index[0].astype(jnp.int32)
    dst = edge_index[1].astype(jnp.int32)
    pad_e = E_PAD - E
    pad_idx = (jnp.arange(pad_e, dtype=jnp.int32) % (N_PAD - N)) + N
    srcp = jnp.concatenate([src, pad_idx]).reshape(32, E_CHUNKS_PER_TILE, E_CHUNK)
    dstp = jnp.concatenate([dst, pad_idx]).reshape(32, E_CHUNKS_PER_TILE, E_CHUNK)
    xp = jnp.concatenate(
        [x.astype(jnp.int32), jnp.zeros((N_PAD - N,), jnp.int32)])
    batch_p = jnp.concatenate(
        [batch.astype(jnp.int32),
         jnp.full((N_PAD - N,), -1, jnp.int32)]).reshape(N_PAD, 1)
    zeros_h = jnp.zeros((N_PAD, H), jnp.float32)
    ones_h = jnp.ones((E_CHUNK, H), jnp.float32)

    h0, degp = _sc_embed_deg(emb, xp, zeros_h, ones_h, dstp)
    g1, dinv = _tc_layer1(h0, degp, W1)
    S1 = _sc_edge_scatter(g1, zeros_h, srcp, dstp)
    g2 = _tc_layer2(S1, g1, dinv, b1.reshape(1, H), W2)
    S2 = _sc_edge_scatter(g2, zeros_h, srcp, dstp)

    linW_p = jnp.pad(linW, ((0, 0), (0, H - NUM_CLASSES)))
    linb_p = jnp.pad(linb, (0, H - NUM_CLASSES)).reshape(1, H)
    out = _tc_final(S2, g2, dinv, b2.reshape(1, H), batch_p, linW_p, linb_p)
    return out[:, :NUM_CLASSES]


# speedup vs baseline: 19.0702x; 1.1755x over previous
import functools

import jax
import jax.numpy as jnp
from jax import lax
from jax.experimental import pallas as pl
from jax.experimental.pallas import tpu as pltpu
from jax.experimental.pallas import tpu_sc as plsc

N = 10000
E = 320000
H = 128
NUM_GRAPHS = 128
NUM_CLASSES = 10

N_PAD = 10240
E_CHUNK = 128
E_CHUNKS_PER_TILE = 80
E_PAD = 32 * E_CHUNKS_PER_TILE * E_CHUNK
ROWS_PER_TILE = N_PAD // 32
ROWS_PER_SUBCORE = N_PAD // 16

_MESH = plsc.VectorSubcoreMesh(core_axis_name="c", subcore_axis_name="s")


@functools.partial(
    pl.kernel,
    out_type=[
        jax.ShapeDtypeStruct((N_PAD, H), jnp.float32),
        jax.ShapeDtypeStruct((2, N_PAD, H), jnp.float32),
    ],
    mesh=_MESH,
    scratch_types=[
        pltpu.VMEM((80,), jnp.int32),
        pltpu.VMEM((80, H), jnp.float32),
        pltpu.VMEM((E_CHUNKS_PER_TILE, E_CHUNK), jnp.int32),
        pltpu.VMEM((E_CHUNK, H), jnp.float32),
        pltpu.VMEM_SHARED((N_PAD, H), jnp.float32),
        pltpu.SemaphoreType.DMA,
    ],
)
def _sc_embed_deg(emb_h, x_h, zeros_h, ones_h, dst_h, h0_h, degp_h,
                  idx_v, rows_v, didx_v, ones_v, acc_s, sem):
    c = lax.axis_index("c")
    s = lax.axis_index("s")
    wid = s * 2 + c

    pltpu.sync_copy(ones_h, ones_v)
    pltpu.sync_copy(zeros_h.at[pl.ds(s * ROWS_PER_SUBCORE, ROWS_PER_SUBCORE)],
                    acc_s.at[pl.ds(s * ROWS_PER_SUBCORE, ROWS_PER_SUBCORE)])
    plsc.subcore_barrier()

    def emb_chunk(t, _):
        base = wid * ROWS_PER_TILE + t * 80
        pltpu.sync_copy(x_h.at[pl.ds(base, 80)], idx_v)
        pltpu.async_copy(emb_h.at[idx_v], rows_v, sem).wait()
        pltpu.sync_copy(rows_v, h0_h.at[pl.ds(base, 80)])
        return 0
    lax.fori_loop(0, ROWS_PER_TILE // 80, emb_chunk, 0)

    pltpu.sync_copy(dst_h.at[wid], didx_v)

    def deg_chunk(j, _):
        pltpu.sync_copy(ones_v, acc_s.at[didx_v.at[j]], add=True)
        return 0
    lax.fori_loop(0, E_CHUNKS_PER_TILE, deg_chunk, 0)
    plsc.subcore_barrier()

    pltpu.sync_copy(
        acc_s.at[pl.ds(s * ROWS_PER_SUBCORE, ROWS_PER_SUBCORE)],
        degp_h.at[c, pl.ds(s * ROWS_PER_SUBCORE, ROWS_PER_SUBCORE)])


@functools.partial(
    pl.kernel,
    out_type=jax.ShapeDtypeStruct((2, N_PAD, H), jnp.float32),
    mesh=_MESH,
    scratch_types=[
        pltpu.VMEM((E_CHUNKS_PER_TILE // 2, E_CHUNK), jnp.int32),
        pltpu.VMEM((E_CHUNKS_PER_TILE // 2, E_CHUNK), jnp.int32),
        pltpu.VMEM((E_CHUNK, H), jnp.float32),
        pltpu.VMEM((E_CHUNK, H), jnp.float32),
        pltpu.VMEM_SHARED((N_PAD, H), jnp.float32),
        pltpu.SemaphoreType.DMA,
        pltpu.SemaphoreType.DMA,
        pltpu.SemaphoreType.DMA,
        pltpu.SemaphoreType.DMA,
    ],
)
def _sc_edge_scatter(g_h, zeros_h, src_h, dst_h, out_h,
                     sidx_v, didx_v, rows0_v, rows1_v, acc_s,
                     sg0, sg1, ss0, ss1):
    c = lax.axis_index("c")
    s = lax.axis_index("s")
    wid = s * 2 + c
    nhc = E_CHUNKS_PER_TILE // 2
    nhalf = nhc // 2

    pltpu.sync_copy(zeros_h.at[pl.ds(s * ROWS_PER_SUBCORE, ROWS_PER_SUBCORE)],
                    acc_s.at[pl.ds(s * ROWS_PER_SUBCORE, ROWS_PER_SUBCORE)])
    plsc.subcore_barrier()

    for half in range(2):
        pltpu.sync_copy(src_h.at[wid, pl.ds(half * nhc, nhc)], sidx_v)
        pltpu.sync_copy(dst_h.at[wid, pl.ds(half * nhc, nhc)], didx_v)

        pltpu.async_copy(g_h.at[sidx_v.at[0]], rows0_v, sg0)

        def pair(j, _):
            e0 = 2 * j
            pltpu.make_async_copy(g_h.at[sidx_v.at[e0]], rows0_v, sg0).wait()

            @pl.when(j > 0)
            def _():
                pltpu.make_async_copy(
                    rows1_v, acc_s.at[didx_v.at[e0 - 1]], ss1).wait()

            pltpu.async_copy(g_h.at[sidx_v.at[e0 + 1]], rows1_v, sg1)
            pltpu.async_copy(rows0_v, acc_s.at[didx_v.at[e0]], ss0, add=True)
            pltpu.make_async_copy(g_h.at[sidx_v.at[e0 + 1]], rows1_v, sg1).wait()

            @pl.when(j < nhalf - 1)
            def _():
                pltpu.make_async_copy(
                    rows0_v, acc_s.at[didx_v.at[e0]], ss0).wait()
                pltpu.async_copy(g_h.at[sidx_v.at[e0 + 2]], rows0_v, sg0)

            pltpu.async_copy(rows1_v, acc_s.at[didx_v.at[e0 + 1]], ss1, add=True)
            return 0
        lax.fori_loop(0, nhalf, pair, 0)
        pltpu.make_async_copy(rows0_v, acc_s.at[didx_v.at[nhc - 2]], ss0).wait()
        pltpu.make_async_copy(rows1_v, acc_s.at[didx_v.at[nhc - 1]], ss1).wait()
    plsc.subcore_barrier()

    pltpu.sync_copy(
        acc_s.at[pl.ds(s * ROWS_PER_SUBCORE, ROWS_PER_SUBCORE)],
        out_h.at[c, pl.ds(s * ROWS_PER_SUBCORE, ROWS_PER_SUBCORE)])


_BLK = 256
_GRID = N_PAD // _BLK


def _tc_layer1_body(h0_r, degp_r, w_r, g_r, dinv_r):
    deg = 1.0 + degp_r[0] + degp_r[1]
    dinv = lax.rsqrt(deg)
    g_r[...] = dinv * jnp.dot(h0_r[...], w_r[...],
                              preferred_element_type=jnp.float32)
    dinv_r[...] = dinv[:, 0:1]


def _tc_layer1(h0, degp, W1):
    return pl.pallas_call(
        _tc_layer1_body,
        grid=(_GRID,),
        in_specs=[
            pl.BlockSpec((_BLK, H), lambda i: (i, 0)),
            pl.BlockSpec((2, _BLK, H), lambda i: (0, i, 0)),
            pl.BlockSpec((H, H), lambda i: (0, 0)),
        ],
        out_specs=[
            pl.BlockSpec((_BLK, H), lambda i: (i, 0)),
            pl.BlockSpec((_BLK, 1), lambda i: (i, 0)),
        ],
        out_shape=[
            jax.ShapeDtypeStruct((N_PAD, H), jnp.float32),
            jax.ShapeDtypeStruct((N_PAD, 1), jnp.float32),
        ],
    )(h0, degp, W1)


def _tc_layer2_body(sp_r, g1_r, dinv_r, b_r, w_r, g2_r):
    h1 = jnp.maximum(
        dinv_r[...] * (sp_r[0] + sp_r[1] + g1_r[...]) + b_r[...], 0.0)
    g2_r[...] = dinv_r[...] * jnp.dot(h1, w_r[...],
                                      preferred_element_type=jnp.float32)


def _tc_layer2(Sp, g1, dinv, b1, W2):
    return pl.pallas_call(
        _tc_layer2_body,
        grid=(_GRID,),
        in_specs=[
            pl.BlockSpec((2, _BLK, H), lambda i: (0, i, 0)),
            pl.BlockSpec((_BLK, H), lambda i: (i, 0)),
            pl.BlockSpec((_BLK, 1), lambda i: (i, 0)),
            pl.BlockSpec((1, H), lambda i: (0, 0)),
            pl.BlockSpec((H, H), lambda i: (0, 0)),
        ],
        out_specs=pl.BlockSpec((_BLK, H), lambda i: (i, 0)),
        out_shape=jax.ShapeDtypeStruct((N_PAD, H), jnp.float32),
    )(Sp, g1, dinv, b1, W2)


def _tc_final_body(sp_r, g2_r, dinv_r, b_r, batch_r, w_r, bb_r, out_r,
                   pooled_s, cnt_s):
    i = pl.program_id(0)

    @pl.when(i == 0)
    def _():
        pooled_s[...] = jnp.zeros((NUM_GRAPHS, H), jnp.float32)
        cnt_s[...] = jnp.zeros((NUM_GRAPHS, H), jnp.float32)

    h2 = jnp.maximum(
        dinv_r[...] * (sp_r[0] + sp_r[1] + g2_r[...]) + b_r[...], 0.0)
    gids = lax.broadcasted_iota(jnp.int32, (_BLK, NUM_GRAPHS), 1)
    onehot = (batch_r[...] == gids).astype(jnp.float32)
    dn = (((0,), (0,)), ((), ()))
    pooled_s[...] += lax.dot_general(onehot, h2, dn,
                                     preferred_element_type=jnp.float32)
    cnt_s[...] += lax.dot_general(onehot, jnp.ones((_BLK, H), jnp.float32),
                                  dn, preferred_element_type=jnp.float32)

    @pl.when(i == _GRID - 1)
    def _():
        pooled = pooled_s[...] / jnp.maximum(cnt_s[...], 1.0)
        out_r[...] = jnp.dot(pooled, w_r[...],
                             preferred_element_type=jnp.float32) + bb_r[...]


def _tc_final(Sp, g2, dinv, b2, batch_p, linW_p, linb_p):
    return pl.pallas_call(
        _tc_final_body,
        grid=(_GRID,),
        in_specs=[
            pl.BlockSpec((2, _BLK, H), lambda i: (0, i, 0)),
            pl.BlockSpec((_BLK, H), lambda i: (i, 0)),
            pl.BlockSpec((_BLK, 1), lambda i: (i, 0)),
            pl.BlockSpec((1, H), lambda i: (0, 0)),
            pl.BlockSpec((_BLK, 1), lambda i: (i, 0)),
            pl.BlockSpec((H, H), lambda i: (0, 0)),
            pl.BlockSpec((1, H), lambda i: (0, 0)),
        ],
        out_specs=pl.BlockSpec((NUM_GRAPHS, H), lambda i: (0, 0)),
        out_shape=jax.ShapeDtypeStruct((NUM_GRAPHS, H), jnp.float32),
        scratch_shapes=[
            pltpu.VMEM((NUM_GRAPHS, H), jnp.float32),
            pltpu.VMEM((NUM_GRAPHS, H), jnp.float32),
        ],
    )(Sp, g2, dinv, b2, batch_p, linW_p, linb_p)


def kernel(x, edge_index, batch, emb, W1, b1, W2, b2, linW, linb):
    src = edge_index[0].astype(jnp.int32)
    dst = edge_index[1].astype(jnp.int32)
    pad_e = E_PAD - E
    pad_idx = (jnp.arange(pad_e, dtype=jnp.int32) % (N_PAD - N)) + N
    srcp = jnp.concatenate([src, pad_idx]).reshape(32, E_CHUNKS_PER_TILE, E_CHUNK)
    dstp = jnp.concatenate([dst, pad_idx]).reshape(32, E_CHUNKS_PER_TILE, E_CHUNK)
    xp = jnp.concatenate(
        [x.astype(jnp.int32), jnp.zeros((N_PAD - N,), jnp.int32)])
    batch_p = jnp.concatenate(
        [batch.astype(jnp.int32),
         jnp.full((N_PAD - N,), -1, jnp.int32)]).reshape(N_PAD, 1)
    zeros_h = jnp.zeros((N_PAD, H), jnp.float32)
    ones_h = jnp.ones((E_CHUNK, H), jnp.float32)

    h0, degp = _sc_embed_deg(emb, xp, zeros_h, ones_h, dstp)
    g1, dinv = _tc_layer1(h0, degp, W1)
    S1 = _sc_edge_scatter(g1, zeros_h, srcp, dstp)
    g2 = _tc_layer2(S1, g1, dinv, b1.reshape(1, H), W2)
    S2 = _sc_edge_scatter(g2, zeros_h, srcp, dstp)

    linW_p = jnp.pad(linW, ((0, 0), (0, H - NUM_CLASSES)))
    linb_p = jnp.pad(linb, (0, H - NUM_CLASSES)).reshape(1, H)
    out = _tc_final(S2, g2, dinv, b2.reshape(1, H), batch_p, linW_p, linb_p)
    return out[:, :NUM_CLASSES]
